# Initial kernel scaffold; baseline (speedup 1.0000x reference)
import jax
import jax.numpy as jnp
from jax.experimental import pallas as pl


def kernel(hv, he, edge_index, mW0, mb0, Wih0, Whh0, bih0, bhh0, mW1, mb1, Wih1, Whh1, bih1, bhh1):
    raise NotImplementedError("write your pallas kernel here")


# speedup vs baseline: 5.1633x; 5.1633x over previous
import jax
import jax.numpy as jnp
from jax import lax
from jax.experimental import pallas as pl
from jax.experimental.pallas import tpu as pltpu
from jax.experimental.pallas import tpu_sc as plsc

N = 10000
E = 320000
H = 128
EH = 16

NC = 2
NS = 16
NW = NC * NS
EPW = 10240
E_PAD = NW * EPW
N_ACC = 10240
RPT = N_ACC // NS
TRASH = N_ACC - 1

CH = 64
GRP = 4
NGRPS = EPW // (GRP * CH)
NSUP = NGRPS // 2

CHA = 64
GRPA = 2
NCHA = EPW // CHA
NGA = NCHA // GRPA

_mesh = plsc.VectorSubcoreMesh(core_axis_name="c", subcore_axis_name="s")


def _fill(ref, nrows, width, value):
  vec = jnp.full((16,), value, jnp.float32)

  def row(i, carry):
    for j in range(width // 16):
      ref[i, pl.ds(j * 16, 16)] = vec
    return carry

  lax.fori_loop(0, nrows, row, 0)


def _worker_id():
  cid = lax.axis_index("c")
  sid = lax.axis_index("s")
  return cid, sid, sid * NC + cid


def _sc_pre_body(he2, dstp, pre_out, idx_d, sb0, sb1, hb0, hb1,
                 es0, es1, acc):
  sbufs = [sb0, sb1]
  hbs = [hb0, hb1]
  esems = [es0, es1]
  cid, sid, wid = _worker_id()
  base = sid * RPT

  pltpu.sync_copy(dstp.at[wid], idx_d)
  _fill(sb0, CHA, 128, 0.0)
  _fill(sb1, CHA, 128, 0.0)
  for k in range(RPT // CHA):
    pltpu.sync_copy(sb0, acc.at[pl.ds(base + k * CHA, CHA)])
  one0 = jnp.where(lax.iota(jnp.int32, 16) == 0, 1.0, 0.0).astype(jnp.float32)
  for sb in sbufs:
    def onerow(i, carry, sb=sb):
      sb[i, pl.ds(16, 16)] = one0
      return carry
    lax.fori_loop(0, CHA, onerow, 0)
  plsc.subcore_barrier()

  def he_src(ci):
    return he2.at[pl.ds(wid * (EPW // 8) + ci * (CHA // 8), CHA // 8)]

  for b in range(GRPA):
    pltpu.async_copy(he_src(b), hbs[b], esems[b])

  def grp(g, carry):
    for b in range(GRPA):
      ci = g * GRPA + b
      pltpu.make_async_copy(he_src(ci), hbs[b], esems[b]).wait()
      for e in range(CHA):
        sbufs[b][e, pl.ds(0, EH)] = hbs[b][e // 8, pl.ds((e % 8) * EH, EH)]
      pltpu.sync_copy(sbufs[b], acc.at[idx_d.at[ci]], add=True)

      @pl.when(g < NGA - 1)
      def _():
        pltpu.async_copy(he_src(ci + GRPA), hbs[b], esems[b])
    return carry

  lax.fori_loop(0, NGA, grp, 0)

  plsc.subcore_barrier()
  for k in range(RPT // CHA):
    sl = pl.ds(base + k * CHA, CHA)
    pltpu.sync_copy(acc.at[sl], sb0)
    pltpu.sync_copy(sb0, pre_out.at[cid, sl])


_sc_pre = pl.kernel(
    _sc_pre_body,
    out_type=jax.ShapeDtypeStruct((NC, N_ACC, 128), jnp.float32),
    mesh=_mesh,
    scratch_types=(
        [pltpu.VMEM((NCHA, CHA), jnp.int32)]
        + [pltpu.VMEM((CHA, 128), jnp.float32)] * 2
        + [pltpu.VMEM((CHA // 8, 128), jnp.float32)] * 2
        + [pltpu.SemaphoreType.DMA] * 2
        + [pltpu.VMEM_SHARED((N_ACC, 128), jnp.float32)]
    ),
)


def _sc_main_body(h, srcp, dstp, hs_out, isrc0, isrc1, idst0, idst1,
                  r0, r1, r2, r3, zb, gs0, gs1, gs2, gs3,
                  iss0, iss1, isd0, isd1, acc):
  isrc = [isrc0, isrc1]
  idst = [idst0, idst1]
  rows = [r0, r1, r2, r3]
  gsem = [gs0, gs1, gs2, gs3]
  isem_s = [iss0, iss1]
  isem_d = [isd0, isd1]
  cid, sid, wid = _worker_id()
  base = sid * RPT

  _fill(zb, CH, H, 0.0)
  for k in range(RPT // CH):
    pltpu.sync_copy(zb, acc.at[pl.ds(base + k * CH, CH)])
  plsc.subcore_barrier()

  pltpu.async_copy(srcp.at[wid, 0], isrc[0], isem_s[0])
  pltpu.async_copy(dstp.at[wid, 0], idst[0], isem_d[0])
  pltpu.make_async_copy(srcp.at[wid, 0], isrc[0], isem_s[0]).wait()
  pltpu.make_async_copy(dstp.at[wid, 0], idst[0], isem_d[0]).wait()
  for b in range(GRP):
    pltpu.async_copy(h.at[isrc[0].at[b]], rows[b], gsem[b])

  def sup(k, carry):
    for off in range(2):
      p = off
      q = 1 - off
      g = 2 * k + off
      live = None if off == 0 else (k < NSUP - 1)

      def run_if_live(fn):
        if live is None:
          fn()
        else:
          pl.when(live)(fn)

      def issue_idx():
        pltpu.async_copy(srcp.at[wid, g + 1], isrc[q], isem_s[q])
        pltpu.async_copy(dstp.at[wid, g + 1], idst[q], isem_d[q])

      run_if_live(issue_idx)

      for b in range(GRP):
        pltpu.make_async_copy(h.at[isrc[p].at[b]], rows[b], gsem[b]).wait()
        pltpu.sync_copy(rows[b], acc.at[idst[p].at[b]], add=True)

        def reissue(b=b):
          if b == 0:
            pltpu.make_async_copy(
                srcp.at[wid, g + 1], isrc[q], isem_s[q]).wait()
            pltpu.make_async_copy(
                dstp.at[wid, g + 1], idst[q], isem_d[q]).wait()
          pltpu.async_copy(h.at[isrc[q].at[b]], rows[b], gsem[b])

        run_if_live(reissue)
    return carry

  lax.fori_loop(0, NSUP, sup, 0)

  plsc.subcore_barrier()
  for k in range(RPT // CH):
    pltpu.sync_copy(acc.at[pl.ds(base + k * CH, CH)], zb)
    pltpu.sync_copy(zb, hs_out.at[cid, pl.ds(base + k * CH, CH)])


_sc_main = pl.kernel(
    _sc_main_body,
    out_type=jax.ShapeDtypeStruct((NC, N_ACC, H), jnp.float32),
    mesh=_mesh,
    scratch_types=(
        [pltpu.VMEM((GRP, CH), jnp.int32)] * 2
        + [pltpu.VMEM((GRP, CH), jnp.int32)] * 2
        + [pltpu.VMEM((CH, H), jnp.float32) for _ in range(GRP)]
        + [pltpu.VMEM((CH, H), jnp.float32)]
        + [pltpu.SemaphoreType.DMA for _ in range(GRP)]
        + [pltpu.SemaphoreType.DMA for _ in range(4)]
        + [pltpu.VMEM_SHARED((N_ACC, H), jnp.float32)]
    ),
)

BN = 1000


def _tc_body(h_ref, hs_ref, pre_ref, mwt_ref, wih_ref, whh_ref,
             mb_ref, bih_ref, bhh_ref, out_ref):
  f32 = jnp.float32
  h = h_ref[...]
  hs = hs_ref[0] + hs_ref[1]
  pre = pre_ref[0] + pre_ref[1]
  she = pre[:, :EH]
  deg = pre[:, EH:EH + 1]
  mwt = mwt_ref[...]
  wd = mwt[0:H]
  ws = mwt[H:2 * H]
  we = mwt[2 * H:2 * H + EH]
  a = (deg * (jnp.dot(h, wd, preferred_element_type=f32) + mb_ref[...])
       + jnp.dot(hs, ws, preferred_element_type=f32)
       + jnp.dot(she, we, preferred_element_type=f32))
  gi = jnp.dot(a, wih_ref[...], preferred_element_type=f32) + bih_ref[...]
  gh = jnp.dot(h, whh_ref[...], preferred_element_type=f32) + bhh_ref[...]
  r = jax.nn.sigmoid(gi[:, :H] + gh[:, :H])
  z = jax.nn.sigmoid(gi[:, H:2 * H] + gh[:, H:2 * H])
  n = jnp.tanh(gi[:, 2 * H:] + r * gh[:, 2 * H:])
  out_ref[...] = (1.0 - z) * n + z * h


_tc_call = pl.pallas_call(
    _tc_body,
    grid=(N // BN,),
    in_specs=[
        pl.BlockSpec((BN, H), lambda i: (i, 0)),
        pl.BlockSpec((NC, BN, H), lambda i: (0, i, 0)),
        pl.BlockSpec((NC, BN, 128), lambda i: (0, i, 0)),
        pl.BlockSpec((2 * H + EH, 2 * H), lambda i: (0, 0)),
        pl.BlockSpec((2 * H, 3 * H), lambda i: (0, 0)),
        pl.BlockSpec((H, 3 * H), lambda i: (0, 0)),
        pl.BlockSpec((1, 2 * H), lambda i: (0, 0)),
        pl.BlockSpec((1, 3 * H), lambda i: (0, 0)),
        pl.BlockSpec((1, 3 * H), lambda i: (0, 0)),
    ],
    out_specs=pl.BlockSpec((BN, H), lambda i: (i, 0)),
    out_shape=jax.ShapeDtypeStruct((N, H), jnp.float32),
)


def kernel(hv, he, edge_index, mW0, mb0, Wih0, Whh0, bih0, bhh0,
           mW1, mb1, Wih1, Whh1, bih1, bhh1):
  pad = E_PAD - E
  src = edge_index[0]
  dst = edge_index[1]
  src_b = jnp.concatenate(
      [src, jnp.zeros((pad,), jnp.int32)]).reshape(NW, NGRPS, GRP, CH)
  dst_flat = jnp.concatenate([dst, jnp.full((pad,), TRASH, jnp.int32)])
  dst_b = dst_flat.reshape(NW, NGRPS, GRP, CH)
  dst_a = dst_flat.reshape(NW, NCHA, CHA)
  he2 = jnp.concatenate(
      [he, jnp.zeros((pad, EH), he.dtype)], axis=0).reshape(E_PAD // 8, 128)

  pre = _sc_pre(he2, dst_a)
  hs0 = _sc_main(hv, src_b, dst_b)
  h1 = _tc_call(hv, hs0, pre, mW0.T, Wih0.T, Whh0.T,
                mb0[None], bih0[None], bhh0[None])
  hs1 = _sc_main(h1, src_b, dst_b)
  h2 = _tc_call(h1, hs1, pre, mW1.T, Wih1.T, Whh1.T,
                mb1[None], bih1[None], bhh1[None])
  return h2


# speedup vs baseline: 5.5616x; 1.0771x over previous
import jax
import jax.numpy as jnp
from jax import lax
from jax.experimental import pallas as pl
from jax.experimental.pallas import tpu as pltpu
from jax.experimental.pallas import tpu_sc as plsc

N = 10000
E = 320000
H = 128
EH = 16

NC = 2
NS = 16
NW = NC * NS
EPW = 10240
E_PAD = NW * EPW
N_ACC = 10240
RPT = N_ACC // NS
TRASH = N_ACC - 1

CH = 64
GRP = 4
NGRPS = EPW // (GRP * CH)
NSUP = NGRPS // 2

CHA = 64
GRPA = 2
NCHA = EPW // CHA
NGA = NCHA // GRPA

_mesh = plsc.VectorSubcoreMesh(core_axis_name="c", subcore_axis_name="s")


def _fill(ref, nrows, width, value):
  vec = jnp.full((16,), value, jnp.float32)

  def row(i, carry):
    for j in range(width // 16):
      ref[i, pl.ds(j * 16, 16)] = vec
    return carry

  lax.fori_loop(0, nrows, row, 0)


def _worker_id():
  cid = lax.axis_index("c")
  sid = lax.axis_index("s")
  return cid, sid, sid * NC + cid


def _sc_pre_body(he2, dstp, pre_out, idx_d, sb0, sb1, hb0, hb1,
                 es0, es1, acc):
  sbufs = [sb0, sb1]
  hbs = [hb0, hb1]
  esems = [es0, es1]
  cid, sid, wid = _worker_id()
  base = sid * RPT

  pltpu.sync_copy(dstp.at[wid], idx_d)
  _fill(sb0, CHA, 128, 0.0)
  _fill(sb1, CHA, 128, 0.0)
  for k in range(RPT // CHA):
    pltpu.sync_copy(sb0, acc.at[pl.ds(base + k * CHA, CHA)])
  one0 = jnp.where(lax.iota(jnp.int32, 16) == 0, 1.0, 0.0).astype(jnp.float32)
  for sb in sbufs:
    def onerow(i, carry, sb=sb):
      sb[i, pl.ds(16, 16)] = one0
      return carry
    lax.fori_loop(0, CHA, onerow, 0)
  plsc.subcore_barrier()

  def he_src(ci):
    return he2.at[pl.ds(wid * (EPW // 8) + ci * (CHA // 8), CHA // 8)]

  for b in range(GRPA):
    pltpu.async_copy(he_src(b), hbs[b], esems[b])

  def grp(g, carry):
    for b in range(GRPA):
      ci = g * GRPA + b
      pltpu.make_async_copy(he_src(ci), hbs[b], esems[b]).wait()
      for e in range(CHA):
        sbufs[b][e, pl.ds(0, EH)] = hbs[b][e // 8, pl.ds((e % 8) * EH, EH)]
      pltpu.sync_copy(sbufs[b], acc.at[idx_d.at[ci]], add=True)

      @pl.when(g < NGA - 1)
      def _():
        pltpu.async_copy(he_src(ci + GRPA), hbs[b], esems[b])
    return carry

  lax.fori_loop(0, NGA, grp, 0)

  plsc.subcore_barrier()
  for k in range(RPT // CHA):
    sl = pl.ds(base + k * CHA, CHA)
    pltpu.sync_copy(acc.at[sl], sb0)
    pltpu.sync_copy(sb0, pre_out.at[cid, sl])


_sc_pre = pl.kernel(
    _sc_pre_body,
    out_type=jax.ShapeDtypeStruct((NC, N_ACC, 128), jnp.float32),
    mesh=_mesh,
    scratch_types=(
        [pltpu.VMEM((NCHA, CHA), jnp.int32)]
        + [pltpu.VMEM((CHA, 128), jnp.float32)] * 2
        + [pltpu.VMEM((CHA // 8, 128), jnp.float32)] * 2
        + [pltpu.SemaphoreType.DMA] * 2
        + [pltpu.VMEM_SHARED((N_ACC, 128), jnp.float32)]
    ),
)


def _sc_main_body(h, srcp, dstp, hs_out, isrc0, isrc1, idst0, idst1,
                  r0, r1, r2, r3, zb, gs0, gs1, gs2, gs3,
                  iss0, iss1, isd0, isd1, acc):
  isrc = [isrc0, isrc1]
  idst = [idst0, idst1]
  rows = [r0, r1, r2, r3]
  gsem = [gs0, gs1, gs2, gs3]
  isem_s = [iss0, iss1]
  isem_d = [isd0, isd1]
  cid, sid, wid = _worker_id()
  base = sid * RPT

  _fill(zb, CH, H, 0.0)
  for k in range(RPT // CH):
    pltpu.sync_copy(zb, acc.at[pl.ds(base + k * CH, CH)])
  plsc.subcore_barrier()

  pltpu.async_copy(srcp.at[wid, 0], isrc[0], isem_s[0])
  pltpu.async_copy(dstp.at[wid, 0], idst[0], isem_d[0])
  pltpu.make_async_copy(srcp.at[wid, 0], isrc[0], isem_s[0]).wait()
  pltpu.make_async_copy(dstp.at[wid, 0], idst[0], isem_d[0]).wait()
  for b in range(GRP):
    pltpu.async_copy(h.at[isrc[0].at[b]], rows[b], gsem[b])

  def sup(k, carry):
    for off in range(2):
      p = off
      q = 1 - off
      g = 2 * k + off
      live = None if off == 0 else (k < NSUP - 1)

      def run_if_live(fn):
        if live is None:
          fn()
        else:
          pl.when(live)(fn)

      def issue_idx():
        pltpu.async_copy(srcp.at[wid, g + 1], isrc[q], isem_s[q])
        pltpu.async_copy(dstp.at[wid, g + 1], idst[q], isem_d[q])

      run_if_live(issue_idx)

      for b in range(GRP):
        pltpu.make_async_copy(h.at[isrc[p].at[b]], rows[b], gsem[b]).wait()
        pltpu.sync_copy(rows[b], acc.at[idst[p].at[b]], add=True)

        def reissue(b=b):
          if b == 0:
            pltpu.make_async_copy(
                srcp.at[wid, g + 1], isrc[q], isem_s[q]).wait()
            pltpu.make_async_copy(
                dstp.at[wid, g + 1], idst[q], isem_d[q]).wait()
          pltpu.async_copy(h.at[isrc[q].at[b]], rows[b], gsem[b])

        run_if_live(reissue)
    return carry

  lax.fori_loop(0, NSUP, sup, 0)

  plsc.subcore_barrier()
  for k in range(RPT // CH):
    pltpu.sync_copy(acc.at[pl.ds(base + k * CH, CH)], zb)
    pltpu.sync_copy(zb, hs_out.at[cid, pl.ds(base + k * CH, CH)])


_sc_main = pl.kernel(
    _sc_main_body,
    out_type=jax.ShapeDtypeStruct((NC, N_ACC, H), jnp.float32),
    mesh=_mesh,
    scratch_types=(
        [pltpu.VMEM((GRP, CH), jnp.int32)] * 2
        + [pltpu.VMEM((GRP, CH), jnp.int32)] * 2
        + [pltpu.VMEM((CH, H), jnp.float32) for _ in range(GRP)]
        + [pltpu.VMEM((CH, H), jnp.float32)]
        + [pltpu.SemaphoreType.DMA for _ in range(GRP)]
        + [pltpu.SemaphoreType.DMA for _ in range(4)]
        + [pltpu.VMEM_SHARED((N_ACC, H), jnp.float32)]
    ),
)

BN = 1000


def _tc_body(h_ref, hs_ref, pre_ref, mwt_ref, wih_ref, whh_ref,
             mb_ref, bih_ref, bhh_ref, out_ref):
  f32 = jnp.float32

  def dot(x, y):
    return jax.lax.dot(x, y, preferred_element_type=f32)

  def dot_hi(x, y):
    return jax.lax.dot(x, y, precision=jax.lax.Precision.HIGHEST,
                       preferred_element_type=f32)

  def rb(x):
    return x.astype(jnp.bfloat16).astype(f32)

  h = h_ref[...]
  hs = hs_ref[0] + hs_ref[1]
  pre = pre_ref[0] + pre_ref[1]
  she = pre[:, :EH]
  deg = pre[:, EH:EH + 1]
  mwt = mwt_ref[...]
  wd = mwt[0:H]
  ws = mwt[H:2 * H]
  we = mwt[2 * H:2 * H + EH]
  a = (deg * (dot(h, wd) + mb_ref[...])
       + dot_hi(hs, rb(ws))
       + dot_hi(she, rb(we)))
  gi = dot(a, wih_ref[...]) + bih_ref[...]
  gh = dot(h, whh_ref[...]) + bhh_ref[...]
  r = jax.nn.sigmoid(gi[:, :H] + gh[:, :H])
  z = jax.nn.sigmoid(gi[:, H:2 * H] + gh[:, H:2 * H])
  n = jnp.tanh(gi[:, 2 * H:] + r * gh[:, 2 * H:])
  out_ref[...] = (1.0 - z) * n + z * h


_tc_call = pl.pallas_call(
    _tc_body,
    grid=(N // BN,),
    in_specs=[
        pl.BlockSpec((BN, H), lambda i: (i, 0)),
        pl.BlockSpec((NC, BN, H), lambda i: (0, i, 0)),
        pl.BlockSpec((NC, BN, 128), lambda i: (0, i, 0)),
        pl.BlockSpec((2 * H + EH, 2 * H), lambda i: (0, 0)),
        pl.BlockSpec((2 * H, 3 * H), lambda i: (0, 0)),
        pl.BlockSpec((H, 3 * H), lambda i: (0, 0)),
        pl.BlockSpec((1, 2 * H), lambda i: (0, 0)),
        pl.BlockSpec((1, 3 * H), lambda i: (0, 0)),
        pl.BlockSpec((1, 3 * H), lambda i: (0, 0)),
    ],
    out_specs=pl.BlockSpec((BN, H), lambda i: (i, 0)),
    out_shape=jax.ShapeDtypeStruct((N, H), jnp.float32),
)


def kernel(hv, he, edge_index, mW0, mb0, Wih0, Whh0, bih0, bhh0,
           mW1, mb1, Wih1, Whh1, bih1, bhh1):
  pad = E_PAD - E
  src = edge_index[0]
  dst = edge_index[1]
  src_b = jnp.concatenate(
      [src, jnp.zeros((pad,), jnp.int32)]).reshape(NW, NGRPS, GRP, CH)
  dst_flat = jnp.concatenate([dst, jnp.full((pad,), TRASH, jnp.int32)])
  dst_b = dst_flat.reshape(NW, NGRPS, GRP, CH)
  dst_a = dst_flat.reshape(NW, NCHA, CHA)
  def rb(x):
    return x.astype(jnp.bfloat16).astype(jnp.float32)

  he2 = jnp.concatenate(
      [rb(he), jnp.zeros((pad, EH), he.dtype)],
      axis=0).reshape(E_PAD // 8, 128)

  pre = _sc_pre(he2, dst_a)
  hs0 = _sc_main(rb(hv), src_b, dst_b)
  h1 = _tc_call(hv, hs0, pre, mW0.T, Wih0.T, Whh0.T,
                mb0[None], bih0[None], bhh0[None])
  hs1 = _sc_main(rb(h1), src_b, dst_b)
  h2 = _tc_call(h1, hs1, pre, mW1.T, Wih1.T, Whh1.T,
                mb1[None], bih1[None], bhh1[None])
  return h2
